# baseline (speedup 1.0000x reference)
import numpy as np
import jax
import jax.numpy as jnp
from jax.experimental import pallas as pl

_HID = 128
_NPATCH = 60
_NNODE = 960
_COUT = 132
_G = 256


def _build_idx4():
    ny, nx = np.meshgrid(np.arange(16), np.arange(60), indexing="ij")
    a = np.stack([nx // 4, ny // 4, nx, ny], axis=-1).astype(np.float32)
    return a.reshape(4, 4, 15, 4, 4)


def _fused(x_ref, w1_ref, b1_ref, w2_ref, b2_ref, idx_ref, out_ref, *, gb):
    x = x_ref[...]
    h = jnp.maximum(
        jnp.dot(x, w1_ref[...], preferred_element_type=jnp.float32)
        + b1_ref[...], 0.0)
    pv = (jnp.dot(h, w2_ref[...], preferred_element_type=jnp.float32)
          + b2_ref[...])
    pv = pv.reshape(gb, 15, 4, _HID).transpose(0, 2, 1, 3)
    t = jnp.broadcast_to(pv[:, :, None, :, None, :],
                         (gb, 4, 4, 15, 4, _HID))
    out_ref[..., : _HID] = t.reshape(gb, _NNODE, _HID)
    out_ref[..., _HID: _HID + 4] = jnp.broadcast_to(
        idx_ref[...].reshape(1, _NNODE, 4), (gb, _NNODE, 4))


def kernel(patch_vectors, W1, b1, W2, b2, Wg1, bg1, Wg2, bg2, mesh_edges):
    del Wg1, bg1, Wg2, bg2, mesh_edges
    gb = 16
    grid = _G // gb
    x = patch_vectors.reshape(_G * _NPATCH, 3)
    idx4 = jnp.asarray(_build_idx4())
    out6 = pl.pallas_call(
        lambda *refs: _fused(*refs, gb=gb),
        grid=(grid,),
        in_specs=[
            pl.BlockSpec((gb * _NPATCH, 3), lambda i: (i, 0)),
            pl.BlockSpec((3, _HID), lambda i: (0, 0)),
            pl.BlockSpec((1, _HID), lambda i: (0, 0)),
            pl.BlockSpec((_HID, _HID), lambda i: (0, 0)),
            pl.BlockSpec((1, _HID), lambda i: (0, 0)),
            pl.BlockSpec((4, 4, 15, 4, 4), lambda i: (0, 0, 0, 0, 0)),
        ],
        out_specs=pl.BlockSpec((gb, _NNODE, 256),
                               lambda i: (i, 0, 0)),
        out_shape=jax.ShapeDtypeStruct((_G, _NNODE, 256), jnp.float32),
    )(x, W1, b1.reshape(1, _HID), W2, b2.reshape(1, _HID), idx4)
    return out6[..., : _COUT]
